# baseline (speedup 1.0000x reference)
import functools

import jax
import jax.numpy as jnp
from jax import lax
from jax.experimental import pallas as pl
from jax.experimental.pallas import tpu as pltpu
from jax.experimental.pallas import tpu_sc as plsc

D = 1024
NH = 16
NKV = 8
HD = 64
DFF = 2048
E = 8
TOPK = 2
EPS = 1e-6
THETA = 10000.0
LAYER_IDX = 0
NEG = -1e30


def _ln_mm_kernel(x_ref, w_ref, wm_ref, o_ref):
    x = x_ref[...]
    var = jnp.mean(x * x, axis=-1, keepdims=True)
    xn = x * jax.lax.rsqrt(var + EPS) * w_ref[...]
    o_ref[...] = jnp.dot(xn, wm_ref[...], preferred_element_type=jnp.float32)


def _ln_matmul(x, w, wm, bs=256):
    s, d = x.shape
    n = wm.shape[1]
    return pl.pallas_call(
        _ln_mm_kernel,
        grid=(s // bs,),
        in_specs=[
            pl.BlockSpec((bs, d), lambda i: (i, 0)),
            pl.BlockSpec((1, d), lambda i: (0, 0)),
            pl.BlockSpec((d, n), lambda i: (0, 0)),
        ],
        out_specs=pl.BlockSpec((bs, n), lambda i: (i, 0)),
        out_shape=jax.ShapeDtypeStruct((s, n), jnp.float32),
    )(x, w.reshape(1, d), wm)


def _rope_apply(x, cos, sin):
    x1 = x[:, :HD // 2]
    x2 = x[:, HD // 2:]
    rot = jnp.concatenate([-x2, x1], axis=-1)
    return x * cos + rot * sin


def _attn_kernel(q_ref, k_ref, v_ref, cq_ref, sq_ref, ck_ref, sk_ref, o_ref,
                 kr_ref, vr_ref, *, bq, s, scale):
    hp = pl.program_id(0)
    i = pl.program_id(1)

    @pl.when(i == 0)
    def _():
        kp = k_ref[...]
        vp = v_ref[...]
        odd = (hp % 2) == 1
        ksel = jnp.where(odd, kp[:, HD:], kp[:, :HD])
        vsel = jnp.where(odd, vp[:, HD:], vp[:, :HD])
        kr_ref[...] = _rope_apply(ksel, ck_ref[...], sk_ref[...])
        vr_ref[...] = vsel

    qp = q_ref[...]
    cq = cq_ref[...]
    sq = sq_ref[...]
    q0 = _rope_apply(qp[:, :HD], cq, sq) * scale
    q1 = _rope_apply(qp[:, HD:], cq, sq) * scale
    rows = i * bq + jax.lax.broadcasted_iota(jnp.int32, (bq, bq), 0)

    def body(j, carry):
        m0, l0, a0, m1, l1, a1 = carry
        kj = kr_ref[pl.ds(j * bq, bq), :]
        vj = vr_ref[pl.ds(j * bq, bq), :]
        cols = j * bq + jax.lax.broadcasted_iota(jnp.int32, (bq, bq), 1)
        mask = cols <= rows

        def upd(q, m, l, a):
            sc = jnp.dot(q, kj.T, preferred_element_type=jnp.float32)
            sc = jnp.where(mask, sc, NEG)
            mn = jnp.maximum(m, jnp.max(sc, axis=-1, keepdims=True))
            p = jnp.exp(sc - mn)
            corr = jnp.exp(m - mn)
            return (mn, l * corr + jnp.sum(p, axis=-1, keepdims=True),
                    a * corr + jnp.dot(p, vj, preferred_element_type=jnp.float32))

        m0, l0, a0 = upd(q0, m0, l0, a0)
        m1, l1, a1 = upd(q1, m1, l1, a1)
        return m0, l0, a0, m1, l1, a1

    mz = jnp.full((bq, 1), NEG, jnp.float32)
    lz = jnp.zeros((bq, 1), jnp.float32)
    az = jnp.zeros((bq, HD), jnp.float32)
    m0, l0, a0, m1, l1, a1 = lax.fori_loop(
        0, i + 1, body, (mz, lz, az, mz, lz, az))
    o_ref[...] = jnp.concatenate([a0 / l0, a1 / l1], axis=1)


def _attention(qkv, cos, sin, bq=256):
    s = qkv.shape[0]
    nbq = s // bq
    kb = NH * HD // 128
    vb = (NH + NKV) * HD // 128
    return pl.pallas_call(
        functools.partial(_attn_kernel, bq=bq, s=s, scale=1.0 / (HD ** 0.5)),
        grid=(NH // 2, nbq),
        in_specs=[
            pl.BlockSpec((bq, 128), lambda hp, i: (i, hp)),
            pl.BlockSpec((s, 128), lambda hp, i: (0, kb + hp // 2)),
            pl.BlockSpec((s, 128), lambda hp, i: (0, vb + hp // 2)),
            pl.BlockSpec((bq, HD), lambda hp, i: (i, 0)),
            pl.BlockSpec((bq, HD), lambda hp, i: (i, 0)),
            pl.BlockSpec((s, HD), lambda hp, i: (0, 0)),
            pl.BlockSpec((s, HD), lambda hp, i: (0, 0)),
        ],
        out_specs=pl.BlockSpec((bq, 128), lambda hp, i: (i, hp)),
        out_shape=jax.ShapeDtypeStruct((s, NH * HD), jnp.float32),
        scratch_shapes=[pltpu.VMEM((s, HD), jnp.float32),
                        pltpu.VMEM((s, HD), jnp.float32)],
    )(qkv, qkv, qkv, cos, sin, cos, sin)


def _proj_kernel(o_ref, wo_ref, res_ref, w2_ref, wr_ref, h_ref, xn_ref, rl_ref):
    h = res_ref[...] + jnp.dot(o_ref[...], wo_ref[...],
                               preferred_element_type=jnp.float32)
    h_ref[...] = h
    var = jnp.mean(h * h, axis=-1, keepdims=True)
    xn = h * jax.lax.rsqrt(var + EPS) * w2_ref[...]
    xn_ref[...] = xn
    rl_ref[...] = jnp.dot(xn, wr_ref[...], preferred_element_type=jnp.float32)


def _proj_res_norm_router(o, wo, res, ln2_w, wr_pad, bs=256):
    s, d = res.shape
    ncol = wr_pad.shape[1]
    return pl.pallas_call(
        _proj_kernel,
        grid=(s // bs,),
        in_specs=[
            pl.BlockSpec((bs, d), lambda i: (i, 0)),
            pl.BlockSpec((d, d), lambda i: (0, 0)),
            pl.BlockSpec((bs, d), lambda i: (i, 0)),
            pl.BlockSpec((1, d), lambda i: (0, 0)),
            pl.BlockSpec((d, ncol), lambda i: (0, 0)),
        ],
        out_specs=[
            pl.BlockSpec((bs, d), lambda i: (i, 0)),
            pl.BlockSpec((bs, d), lambda i: (i, 0)),
            pl.BlockSpec((bs, ncol), lambda i: (i, 0)),
        ],
        out_shape=[
            jax.ShapeDtypeStruct((s, d), jnp.float32),
            jax.ShapeDtypeStruct((s, d), jnp.float32),
            jax.ShapeDtypeStruct((s, ncol), jnp.float32),
        ],
    )(o, wo, res, ln2_w.reshape(1, d), wr_pad)


_NC = 2
_NS = 16
_NW = _NC * _NS


def _sc_gather_rows(table, idx):
    n = idx.shape[0]
    d = table.shape[1]
    per_w = n // _NW
    chunk = per_w
    for c in (64, 56, 48, 40, 32, 24, 16, 8):
        if per_w % c == 0 and 2 * c * d * 4 + per_w * 4 + 4096 <= 500_000:
            chunk = c
            break
    nch = per_w // chunk
    mesh = plsc.VectorSubcoreMesh(core_axis_name="c", subcore_axis_name="s")

    @functools.partial(
        pl.kernel,
        mesh=mesh,
        out_type=jax.ShapeDtypeStruct((n, d), jnp.float32),
        scratch_types=[
            pltpu.VMEM((per_w,), jnp.int32),
            pltpu.VMEM((chunk, d), jnp.float32),
            pltpu.VMEM((chunk, d), jnp.float32),
            pltpu.SemaphoreType.DMA,
            pltpu.SemaphoreType.DMA,
            pltpu.SemaphoreType.DMA,
            pltpu.SemaphoreType.DMA,
        ],
    )
    def k(table_hbm, idx_hbm, out_hbm, idx_v, buf0, buf1, sg0, sg1, so0, so1):
        wid = lax.axis_index("s") * _NC + lax.axis_index("c")
        base = wid * per_w
        bufs = (buf0, buf1)
        sg = (sg0, sg1)
        so = (so0, so1)
        pltpu.sync_copy(idx_hbm.at[pl.ds(base, per_w)], idx_v)

        def gather(c):
            return pltpu.async_copy(
                table_hbm.at[idx_v.at[pl.ds(c * chunk, chunk)]],
                bufs[c % 2], sg[c % 2])

        def put(c):
            return pltpu.async_copy(
                bufs[c % 2], out_hbm.at[pl.ds(base + c * chunk, chunk)],
                so[c % 2])

        hg = {0: gather(0)}
        ho = {}
        for c in range(nch):
            if c + 1 < nch:
                if c - 1 >= 0:
                    ho[c - 1].wait()
                hg[c + 1] = gather(c + 1)
            hg[c].wait()
            ho[c] = put(c)
        if nch >= 2:
            ho[nch - 2].wait()
        ho[nch - 1].wait()

    return k(table, idx)


def _group_ffn_kernel(meta_ref, xs_ref, w1_ref, w3_ref, w2_ref, ys_ref):
    b = pl.program_id(0)

    @pl.when(meta_ref[1, b] == 1)
    def _():
        x = xs_ref[...]
        a = jnp.dot(x, w1_ref[0], preferred_element_type=jnp.float32)
        c = jnp.dot(x, w3_ref[0], preferred_element_type=jnp.float32)
        g = (a * jax.nn.sigmoid(a)) * c
        ys_ref[...] = jnp.dot(g, w2_ref[0], preferred_element_type=jnp.float32)


def _group_ffn(meta, xs, w1, w3, w2, bt, nblk):
    cap, d = xs.shape
    _, _, dff = w1.shape
    grid_spec = pltpu.PrefetchScalarGridSpec(
        num_scalar_prefetch=1,
        grid=(nblk,),
        in_specs=[
            pl.BlockSpec((bt, d), lambda b, m: (b, 0)),
            pl.BlockSpec((1, d, dff), lambda b, m: (m[0, b], 0, 0)),
            pl.BlockSpec((1, d, dff), lambda b, m: (m[0, b], 0, 0)),
            pl.BlockSpec((1, dff, d), lambda b, m: (m[0, b], 0, 0)),
        ],
        out_specs=pl.BlockSpec((bt, d), lambda b, m: (b, 0)),
    )
    return pl.pallas_call(
        _group_ffn_kernel,
        grid_spec=grid_spec,
        out_shape=jax.ShapeDtypeStruct((cap, d), jnp.float32),
    )(meta, xs, w1, w3, w2)


def _combine_kernel(h2_ref, ya_ref, yb_ref, tw_ref, o_ref):
    wa = tw_ref[:, 0:1]
    wb = tw_ref[:, 1:2]
    o_ref[...] = h2_ref[...] + wa * ya_ref[...] + wb * yb_ref[...]


def _combine(h2, yg, tw_pad, bs=512):
    s, d = h2.shape
    nb = s // bs
    return pl.pallas_call(
        _combine_kernel,
        grid=(nb,),
        in_specs=[
            pl.BlockSpec((bs, d), lambda i: (i, 0)),
            pl.BlockSpec((bs, d), lambda i: (i, 0)),
            pl.BlockSpec((bs, d), lambda i: (nb + i, 0)),
            pl.BlockSpec((bs, 128), lambda i: (i, 0)),
        ],
        out_specs=pl.BlockSpec((bs, d), lambda i: (i, 0)),
        out_shape=jax.ShapeDtypeStruct((s, d), jnp.float32),
    )(h2, yg, yg, tw_pad)


def _moe_kernel(x_ref, dw_ref, h2_ref, w1_ref, w3_ref, w2_ref, o_ref, *, bt):
    e = pl.program_id(0)
    f = pl.program_id(1)
    t = pl.program_id(2)
    x = x_ref[...]
    a = jnp.dot(x, w1_ref[0], preferred_element_type=jnp.float32)
    c = jnp.dot(x, w3_ref[0], preferred_element_type=jnp.float32)
    g = (a * jax.nn.sigmoid(a)) * c
    y = jnp.dot(g, w2_ref[0], preferred_element_type=jnp.float32)
    lane = jax.lax.broadcasted_iota(jnp.int32, dw_ref.shape, 1)
    wcol = jnp.sum(jnp.where(lane == e, dw_ref[...], 0.0), axis=1,
                   keepdims=True)
    contrib = wcol * y
    sl = pl.ds(t * bt, bt)

    @pl.when((e == 0) & (f == 0))
    def _init():
        o_ref[sl, :] = h2_ref[...] + contrib

    @pl.when((e != 0) | (f != 0))
    def _acc():
        o_ref[sl, :] = o_ref[sl, :] + contrib


def _moe(xn, dw_pad, h2, w1, w3, w2, bt=256, bf=1024):
    s, d = xn.shape
    e, _, dff = w1.shape
    ncol = dw_pad.shape[1]
    return pl.pallas_call(
        functools.partial(_moe_kernel, bt=bt),
        grid=(e, dff // bf, s // bt),
        in_specs=[
            pl.BlockSpec((bt, d), lambda ei, f, t: (t, 0)),
            pl.BlockSpec((bt, ncol), lambda ei, f, t: (t, 0)),
            pl.BlockSpec((bt, d), lambda ei, f, t: (t, 0)),
            pl.BlockSpec((1, d, bf), lambda ei, f, t: (ei, 0, f)),
            pl.BlockSpec((1, d, bf), lambda ei, f, t: (ei, 0, f)),
            pl.BlockSpec((1, bf, d), lambda ei, f, t: (ei, f, 0)),
        ],
        out_specs=pl.BlockSpec((s, d), lambda ei, f, t: (0, 0)),
        out_shape=jax.ShapeDtypeStruct((s, d), jnp.float32),
    )(xn, dw_pad, h2, w1, w3, w2)


def kernel(hidden_states, position_ids, gate_logits, ln1_w, ln2_w, wq, wk, wv,
           wo, w_router, w1, w3, w2):
    b, s, d = hidden_states.shape
    x = hidden_states.reshape(s, d)

    wqkv = jnp.concatenate([wq, wk, wv], axis=1)
    qkv = _ln_matmul(x, ln1_w, wqkv)

    pos = position_ids.reshape(s).astype(jnp.float32)
    inv = 1.0 / (THETA ** (jnp.arange(0, HD, 2, dtype=jnp.float32) / HD))
    freqs = pos[:, None] * inv
    emb = jnp.concatenate([freqs, freqs], axis=-1)
    cos = jnp.cos(emb)
    sin = jnp.sin(emb)

    o = _attention(qkv, cos, sin)

    wr_pad = jnp.zeros((d, 128), jnp.float32).at[:, :E].set(w_router)
    h2, xn, rl_pad = _proj_res_norm_router(o, wo, x, ln2_w, wr_pad)
    router_logits = rl_pad[:, :E]

    probs = jax.nn.softmax(router_logits, axis=-1)
    topw, topi = jax.lax.top_k(probs, TOPK)
    topw = topw / jnp.sum(topw, axis=-1, keepdims=True)

    bt = 128
    nblk = (s * TOPK) // bt + E
    cap = nblk * bt
    npair = s * TOPK
    ef = topi.reshape(npair).astype(jnp.int32)
    onehot = (ef[:, None] == jnp.arange(E, dtype=jnp.int32)[None, :]
              ).astype(jnp.int32)
    cum = jnp.cumsum(onehot, axis=0)
    rank = jnp.sum(onehot * cum, axis=1) - 1
    counts = cum[-1]
    blocks = (counts + bt - 1) // bt
    cblocks = jnp.cumsum(blocks).astype(jnp.int32)
    pad_start = bt * jnp.concatenate(
        [jnp.zeros((1,), jnp.int32), cblocks[:-1]])
    pad_pos = pad_start[ef] + rank
    token_of_pair = jnp.arange(npair, dtype=jnp.int32) // TOPK
    src_token = (jnp.arange(cap, dtype=jnp.int32) % s).at[pad_pos].set(
        token_of_pair)
    posAB = jnp.concatenate(
        [pad_pos.reshape(-1, TOPK)[:, 0], pad_pos.reshape(-1, TOPK)[:, 1]])
    nb_used = cblocks[-1]
    bid = jnp.arange(nblk, dtype=jnp.int32)
    bclamp = jnp.minimum(bid, nb_used - 1)
    block_expert = jnp.searchsorted(cblocks, bclamp, side='right')
    meta = jnp.stack([block_expert.astype(jnp.int32),
                      (bid < nb_used).astype(jnp.int32)])

    xs = _sc_gather_rows(xn, src_token)
    ys = _group_ffn(meta, xs, w1, w3, w2, bt, nblk)
    yg = _sc_gather_rows(ys, posAB)
    tw_pad = jnp.zeros((s, 128), jnp.float32).at[:, :TOPK].set(topw)
    h_out = _combine(h2, yg, tw_pad)

    gate_logits = gate_logits.at[LAYER_IDX].set(router_logits)
    return (h_out.reshape(b, s, d), position_ids, gate_logits)


# speedup vs baseline: 1.1561x; 1.1561x over previous
import functools

import jax
import jax.numpy as jnp
from jax import lax
from jax.experimental import pallas as pl
from jax.experimental.pallas import tpu as pltpu
from jax.experimental.pallas import tpu_sc as plsc

D = 1024
NH = 16
NKV = 8
HD = 64
DFF = 2048
E = 8
TOPK = 2
EPS = 1e-6
THETA = 10000.0
LAYER_IDX = 0
NEG = -1e30


def _ln_mm_kernel(x_ref, w_ref, wm_ref, o_ref):
    x = x_ref[...]
    var = jnp.mean(x * x, axis=-1, keepdims=True)
    xn = x * jax.lax.rsqrt(var + EPS) * w_ref[...]
    o_ref[...] = jnp.dot(xn, wm_ref[...], preferred_element_type=jnp.float32)


def _ln_matmul(x, w, wm, bs=256):
    s, d = x.shape
    n = wm.shape[1]
    return pl.pallas_call(
        _ln_mm_kernel,
        grid=(s // bs,),
        in_specs=[
            pl.BlockSpec((bs, d), lambda i: (i, 0)),
            pl.BlockSpec((1, d), lambda i: (0, 0)),
            pl.BlockSpec((d, n), lambda i: (0, 0)),
        ],
        out_specs=pl.BlockSpec((bs, n), lambda i: (i, 0)),
        out_shape=jax.ShapeDtypeStruct((s, n), jnp.float32),
    )(x, w.reshape(1, d), wm)


def _rope_apply(x, cos, sin):
    x1 = x[:, :HD // 2]
    x2 = x[:, HD // 2:]
    rot = jnp.concatenate([-x2, x1], axis=-1)
    return x * cos + rot * sin


def _attn_kernel(q_ref, k_ref, v_ref, cq_ref, sq_ref, ck_ref, sk_ref, o_ref,
                 kr_ref, vr_ref, *, bq, s, scale):
    hp = pl.program_id(0)
    i = pl.program_id(1)

    @pl.when(i == 0)
    def _():
        kp = k_ref[...]
        vp = v_ref[...]
        odd = (hp % 2) == 1
        ksel = jnp.where(odd, kp[:, HD:], kp[:, :HD])
        vsel = jnp.where(odd, vp[:, HD:], vp[:, :HD])
        kr_ref[...] = _rope_apply(ksel, ck_ref[...], sk_ref[...])
        vr_ref[...] = vsel

    qp = q_ref[...]
    cq = cq_ref[...]
    sq = sq_ref[...]
    q0 = _rope_apply(qp[:, :HD], cq, sq) * scale
    q1 = _rope_apply(qp[:, HD:], cq, sq) * scale
    q01 = jnp.concatenate([q0, q1], axis=0)
    rows = i * bq + jax.lax.broadcasted_iota(jnp.int32, (bq, bq), 0)

    def body(j, carry):
        m, l, acc = carry
        kj = kr_ref[pl.ds(j * bq, bq), :]
        vj = vr_ref[pl.ds(j * bq, bq), :]
        cols = j * bq + jax.lax.broadcasted_iota(jnp.int32, (bq, bq), 1)
        mask = cols <= rows
        mask2 = jnp.concatenate([mask, mask], axis=0)
        sc = lax.dot_general(q01, kj, (((1,), (1,)), ((), ())),
                             preferred_element_type=jnp.float32)
        sc = jnp.where(mask2, sc, NEG)
        mn = jnp.maximum(m, jnp.max(sc, axis=-1, keepdims=True))
        p = jnp.exp(sc - mn)
        corr = jnp.exp(m - mn)
        l = l * corr + jnp.sum(p, axis=-1, keepdims=True)
        acc = acc * corr + jnp.dot(p, vj, preferred_element_type=jnp.float32)
        return mn, l, acc

    mz = jnp.full((2 * bq, 1), NEG, jnp.float32)
    lz = jnp.zeros((2 * bq, 1), jnp.float32)
    az = jnp.zeros((2 * bq, HD), jnp.float32)
    m, l, acc = lax.fori_loop(0, i + 1, body, (mz, lz, az))
    o = acc / l
    o_ref[...] = jnp.concatenate([o[:bq], o[bq:]], axis=1)


def _attention(qkv, cos, sin, bq=512):
    s = qkv.shape[0]
    nbq = s // bq
    kb = NH * HD // 128
    vb = (NH + NKV) * HD // 128
    return pl.pallas_call(
        functools.partial(_attn_kernel, bq=bq, s=s, scale=1.0 / (HD ** 0.5)),
        grid=(NH // 2, nbq),
        in_specs=[
            pl.BlockSpec((bq, 128), lambda hp, i: (i, hp)),
            pl.BlockSpec((s, 128), lambda hp, i: (0, kb + hp // 2)),
            pl.BlockSpec((s, 128), lambda hp, i: (0, vb + hp // 2)),
            pl.BlockSpec((bq, HD), lambda hp, i: (i, 0)),
            pl.BlockSpec((bq, HD), lambda hp, i: (i, 0)),
            pl.BlockSpec((s, HD), lambda hp, i: (0, 0)),
            pl.BlockSpec((s, HD), lambda hp, i: (0, 0)),
        ],
        out_specs=pl.BlockSpec((bq, 128), lambda hp, i: (i, hp)),
        out_shape=jax.ShapeDtypeStruct((s, NH * HD), jnp.float32),
        scratch_shapes=[pltpu.VMEM((s, HD), jnp.float32),
                        pltpu.VMEM((s, HD), jnp.float32)],
    )(qkv, qkv, qkv, cos, sin, cos, sin)


def _proj_kernel(o_ref, wo_ref, res_ref, w2_ref, wr_ref, h_ref, xn_ref, rl_ref):
    h = res_ref[...] + jnp.dot(o_ref[...], wo_ref[...],
                               preferred_element_type=jnp.float32)
    h_ref[...] = h
    var = jnp.mean(h * h, axis=-1, keepdims=True)
    xn = h * jax.lax.rsqrt(var + EPS) * w2_ref[...]
    xn_ref[...] = xn
    rl_ref[...] = jnp.dot(xn, wr_ref[...], preferred_element_type=jnp.float32)


def _proj_res_norm_router(o, wo, res, ln2_w, wr_pad, bs=256):
    s, d = res.shape
    ncol = wr_pad.shape[1]
    return pl.pallas_call(
        _proj_kernel,
        grid=(s // bs,),
        in_specs=[
            pl.BlockSpec((bs, d), lambda i: (i, 0)),
            pl.BlockSpec((d, d), lambda i: (0, 0)),
            pl.BlockSpec((bs, d), lambda i: (i, 0)),
            pl.BlockSpec((1, d), lambda i: (0, 0)),
            pl.BlockSpec((d, ncol), lambda i: (0, 0)),
        ],
        out_specs=[
            pl.BlockSpec((bs, d), lambda i: (i, 0)),
            pl.BlockSpec((bs, d), lambda i: (i, 0)),
            pl.BlockSpec((bs, ncol), lambda i: (i, 0)),
        ],
        out_shape=[
            jax.ShapeDtypeStruct((s, d), jnp.float32),
            jax.ShapeDtypeStruct((s, d), jnp.float32),
            jax.ShapeDtypeStruct((s, ncol), jnp.float32),
        ],
    )(o, wo, res, ln2_w.reshape(1, d), wr_pad)


_NC = 2
_NS = 16
_NW = _NC * _NS


def _sc_gather_rows(table, idx):
    n = idx.shape[0]
    d = table.shape[1]
    per_w = n // _NW
    chunk = per_w
    for c in (64, 56, 48, 40, 32, 24, 16, 8):
        if per_w % c == 0 and 2 * c * d * 4 + per_w * 4 + 4096 <= 500_000:
            chunk = c
            break
    nch = per_w // chunk
    mesh = plsc.VectorSubcoreMesh(core_axis_name="c", subcore_axis_name="s")

    @functools.partial(
        pl.kernel,
        mesh=mesh,
        out_type=jax.ShapeDtypeStruct((n, d), jnp.float32),
        scratch_types=[
            pltpu.VMEM((per_w,), jnp.int32),
            pltpu.VMEM((chunk, d), jnp.float32),
            pltpu.VMEM((chunk, d), jnp.float32),
            pltpu.SemaphoreType.DMA,
            pltpu.SemaphoreType.DMA,
            pltpu.SemaphoreType.DMA,
            pltpu.SemaphoreType.DMA,
        ],
    )
    def k(table_hbm, idx_hbm, out_hbm, idx_v, buf0, buf1, sg0, sg1, so0, so1):
        wid = lax.axis_index("s") * _NC + lax.axis_index("c")
        base = wid * per_w
        bufs = (buf0, buf1)
        sg = (sg0, sg1)
        so = (so0, so1)
        pltpu.sync_copy(idx_hbm.at[pl.ds(base, per_w)], idx_v)

        def gather(c):
            return pltpu.async_copy(
                table_hbm.at[idx_v.at[pl.ds(c * chunk, chunk)]],
                bufs[c % 2], sg[c % 2])

        def put(c):
            return pltpu.async_copy(
                bufs[c % 2], out_hbm.at[pl.ds(base + c * chunk, chunk)],
                so[c % 2])

        hg = {0: gather(0)}
        ho = {}
        for c in range(nch):
            if c + 1 < nch:
                if c - 1 >= 0:
                    ho[c - 1].wait()
                hg[c + 1] = gather(c + 1)
            hg[c].wait()
            ho[c] = put(c)
        if nch >= 2:
            ho[nch - 2].wait()
        ho[nch - 1].wait()

    return k(table, idx)


def _group_ffn_kernel(meta_ref, xs_ref, w1_ref, w3_ref, w2_ref, ys_ref):
    b = pl.program_id(0)

    @pl.when(meta_ref[1, b] == 1)
    def _():
        x = xs_ref[...]
        a = jnp.dot(x, w1_ref[0], preferred_element_type=jnp.float32)
        c = jnp.dot(x, w3_ref[0], preferred_element_type=jnp.float32)
        g = (a * jax.nn.sigmoid(a)) * c
        ys_ref[...] = jnp.dot(g, w2_ref[0], preferred_element_type=jnp.float32)


def _group_ffn(meta, xs, w1, w3, w2, bt, nblk):
    cap, d = xs.shape
    _, _, dff = w1.shape
    grid_spec = pltpu.PrefetchScalarGridSpec(
        num_scalar_prefetch=1,
        grid=(nblk,),
        in_specs=[
            pl.BlockSpec((bt, d), lambda b, m: (b, 0)),
            pl.BlockSpec((1, d, dff), lambda b, m: (m[0, b], 0, 0)),
            pl.BlockSpec((1, d, dff), lambda b, m: (m[0, b], 0, 0)),
            pl.BlockSpec((1, dff, d), lambda b, m: (m[0, b], 0, 0)),
        ],
        out_specs=pl.BlockSpec((bt, d), lambda b, m: (b, 0)),
    )
    return pl.pallas_call(
        _group_ffn_kernel,
        grid_spec=grid_spec,
        out_shape=jax.ShapeDtypeStruct((cap, d), jnp.float32),
    )(meta, xs, w1, w3, w2)


def _combine_kernel(h2_ref, ya_ref, yb_ref, tw_ref, o_ref):
    wa = tw_ref[:, 0:1]
    wb = tw_ref[:, 1:2]
    o_ref[...] = h2_ref[...] + wa * ya_ref[...] + wb * yb_ref[...]


def _combine(h2, yg, tw_pad, bs=512):
    s, d = h2.shape
    nb = s // bs
    return pl.pallas_call(
        _combine_kernel,
        grid=(nb,),
        in_specs=[
            pl.BlockSpec((bs, d), lambda i: (i, 0)),
            pl.BlockSpec((bs, d), lambda i: (i, 0)),
            pl.BlockSpec((bs, d), lambda i: (nb + i, 0)),
            pl.BlockSpec((bs, 128), lambda i: (i, 0)),
        ],
        out_specs=pl.BlockSpec((bs, d), lambda i: (i, 0)),
        out_shape=jax.ShapeDtypeStruct((s, d), jnp.float32),
    )(h2, yg, yg, tw_pad)


def _moe_kernel(x_ref, dw_ref, h2_ref, w1_ref, w3_ref, w2_ref, o_ref, *, bt):
    e = pl.program_id(0)
    f = pl.program_id(1)
    t = pl.program_id(2)
    x = x_ref[...]
    a = jnp.dot(x, w1_ref[0], preferred_element_type=jnp.float32)
    c = jnp.dot(x, w3_ref[0], preferred_element_type=jnp.float32)
    g = (a * jax.nn.sigmoid(a)) * c
    y = jnp.dot(g, w2_ref[0], preferred_element_type=jnp.float32)
    lane = jax.lax.broadcasted_iota(jnp.int32, dw_ref.shape, 1)
    wcol = jnp.sum(jnp.where(lane == e, dw_ref[...], 0.0), axis=1,
                   keepdims=True)
    contrib = wcol * y
    sl = pl.ds(t * bt, bt)

    @pl.when((e == 0) & (f == 0))
    def _init():
        o_ref[sl, :] = h2_ref[...] + contrib

    @pl.when((e != 0) | (f != 0))
    def _acc():
        o_ref[sl, :] = o_ref[sl, :] + contrib


def _moe(xn, dw_pad, h2, w1, w3, w2, bt=256, bf=1024):
    s, d = xn.shape
    e, _, dff = w1.shape
    ncol = dw_pad.shape[1]
    return pl.pallas_call(
        functools.partial(_moe_kernel, bt=bt),
        grid=(e, dff // bf, s // bt),
        in_specs=[
            pl.BlockSpec((bt, d), lambda ei, f, t: (t, 0)),
            pl.BlockSpec((bt, ncol), lambda ei, f, t: (t, 0)),
            pl.BlockSpec((bt, d), lambda ei, f, t: (t, 0)),
            pl.BlockSpec((1, d, bf), lambda ei, f, t: (ei, 0, f)),
            pl.BlockSpec((1, d, bf), lambda ei, f, t: (ei, 0, f)),
            pl.BlockSpec((1, bf, d), lambda ei, f, t: (ei, f, 0)),
        ],
        out_specs=pl.BlockSpec((s, d), lambda ei, f, t: (0, 0)),
        out_shape=jax.ShapeDtypeStruct((s, d), jnp.float32),
    )(xn, dw_pad, h2, w1, w3, w2)


def kernel(hidden_states, position_ids, gate_logits, ln1_w, ln2_w, wq, wk, wv,
           wo, w_router, w1, w3, w2):
    b, s, d = hidden_states.shape
    x = hidden_states.reshape(s, d)

    wqkv = jnp.concatenate([wq, wk, wv], axis=1)
    qkv = _ln_matmul(x, ln1_w, wqkv)

    pos = position_ids.reshape(s).astype(jnp.float32)
    inv = 1.0 / (THETA ** (jnp.arange(0, HD, 2, dtype=jnp.float32) / HD))
    freqs = pos[:, None] * inv
    emb = jnp.concatenate([freqs, freqs], axis=-1)
    cos = jnp.cos(emb)
    sin = jnp.sin(emb)

    o = _attention(qkv, cos, sin)

    wr_pad = jnp.zeros((d, 128), jnp.float32).at[:, :E].set(w_router)
    h2, xn, rl_pad = _proj_res_norm_router(o, wo, x, ln2_w, wr_pad)
    router_logits = rl_pad[:, :E]

    probs = jax.nn.softmax(router_logits, axis=-1)
    topw, topi = jax.lax.top_k(probs, TOPK)
    topw = topw / jnp.sum(topw, axis=-1, keepdims=True)

    bt = 128
    nblk = (s * TOPK) // bt + E
    cap = nblk * bt
    npair = s * TOPK
    ef = topi.reshape(npair).astype(jnp.int32)
    onehot = (ef[:, None] == jnp.arange(E, dtype=jnp.int32)[None, :]
              ).astype(jnp.int32)
    cum = jnp.cumsum(onehot, axis=0)
    rank = jnp.sum(onehot * cum, axis=1) - 1
    counts = cum[-1]
    blocks = (counts + bt - 1) // bt
    cblocks = jnp.cumsum(blocks).astype(jnp.int32)
    pad_start = bt * jnp.concatenate(
        [jnp.zeros((1,), jnp.int32), cblocks[:-1]])
    pad_pos = pad_start[ef] + rank
    token_of_pair = jnp.arange(npair, dtype=jnp.int32) // TOPK
    src_token = (jnp.arange(cap, dtype=jnp.int32) % s).at[pad_pos].set(
        token_of_pair)
    posAB = jnp.concatenate(
        [pad_pos.reshape(-1, TOPK)[:, 0], pad_pos.reshape(-1, TOPK)[:, 1]])
    nb_used = cblocks[-1]
    bid = jnp.arange(nblk, dtype=jnp.int32)
    bclamp = jnp.minimum(bid, nb_used - 1)
    block_expert = jnp.searchsorted(cblocks, bclamp, side='right')
    meta = jnp.stack([block_expert.astype(jnp.int32),
                      (bid < nb_used).astype(jnp.int32)])

    xs = _sc_gather_rows(xn, src_token)
    ys = _group_ffn(meta, xs, w1, w3, w2, bt, nblk)
    yg = _sc_gather_rows(ys, posAB)
    tw_pad = jnp.zeros((s, 128), jnp.float32).at[:, :TOPK].set(topw)
    h_out = _combine(h2, yg, tw_pad)

    gate_logits = gate_logits.at[LAYER_IDX].set(router_logits)
    return (h_out.reshape(b, s, d), position_ids, gate_logits)


# speedup vs baseline: 1.1968x; 1.0352x over previous
import functools

import jax
import jax.numpy as jnp
from jax import lax
from jax.experimental import pallas as pl
from jax.experimental.pallas import tpu as pltpu
from jax.experimental.pallas import tpu_sc as plsc

D = 1024
NH = 16
NKV = 8
HD = 64
DFF = 2048
E = 8
TOPK = 2
EPS = 1e-6
THETA = 10000.0
LAYER_IDX = 0
NEG = -1e30


def _ln_mm_kernel(x_ref, w_ref, wm_ref, o_ref):
    x = x_ref[...]
    var = jnp.mean(x * x, axis=-1, keepdims=True)
    xn = x * jax.lax.rsqrt(var + EPS) * w_ref[...]
    o_ref[...] = jnp.dot(xn, wm_ref[...], preferred_element_type=jnp.float32)


def _ln_matmul(x, w, wm, bs=256):
    s, d = x.shape
    n = wm.shape[1]
    return pl.pallas_call(
        _ln_mm_kernel,
        grid=(s // bs,),
        in_specs=[
            pl.BlockSpec((bs, d), lambda i: (i, 0)),
            pl.BlockSpec((1, d), lambda i: (0, 0)),
            pl.BlockSpec((d, n), lambda i: (0, 0)),
        ],
        out_specs=pl.BlockSpec((bs, n), lambda i: (i, 0)),
        out_shape=jax.ShapeDtypeStruct((s, n), jnp.float32),
    )(x, w.reshape(1, d), wm)


def _rope_apply(x, cos, sin):
    x1 = x[:, :HD // 2]
    x2 = x[:, HD // 2:]
    rot = jnp.concatenate([-x2, x1], axis=-1)
    return x * cos + rot * sin


def _attn_kernel(q_ref, k_ref, v_ref, cq_ref, sq_ref, ck_ref, sk_ref, o_ref,
                 kr_ref, vr_ref, *, bq, s, scale):
    hp = pl.program_id(0)
    i = pl.program_id(1)

    @pl.when(i == 0)
    def _():
        kp = k_ref[...]
        vp = v_ref[...]
        odd = (hp % 2) == 1
        ksel = jnp.where(odd, kp[:, HD:], kp[:, :HD])
        vsel = jnp.where(odd, vp[:, HD:], vp[:, :HD])
        kr_ref[...] = _rope_apply(ksel, ck_ref[...], sk_ref[...])
        vr_ref[...] = vsel

    qp = q_ref[...]
    cq = cq_ref[...]
    sq = sq_ref[...]
    q0 = _rope_apply(qp[:, :HD], cq, sq) * scale
    q1 = _rope_apply(qp[:, HD:], cq, sq) * scale
    q01 = jnp.concatenate([q0, q1], axis=0)

    def chunk(j, carry, mask2):
        m, l, acc = carry
        kj = kr_ref[pl.ds(j * bq, bq), :]
        vj = vr_ref[pl.ds(j * bq, bq), :]
        sc = lax.dot_general(q01, kj, (((1,), (1,)), ((), ())),
                             preferred_element_type=jnp.float32)
        if mask2 is not None:
            sc = jnp.where(mask2, sc, NEG)
        mn = jnp.maximum(m, jnp.max(sc, axis=-1, keepdims=True))
        p = jnp.exp(sc - mn)
        corr = jnp.exp(m - mn)
        l = l * corr + jnp.sum(p, axis=-1, keepdims=True)
        acc = acc * corr + jnp.dot(p, vj, preferred_element_type=jnp.float32)
        return mn, l, acc

    mz = jnp.full((2 * bq, 1), NEG, jnp.float32)
    lz = jnp.zeros((2 * bq, 1), jnp.float32)
    az = jnp.zeros((2 * bq, HD), jnp.float32)
    carry = lax.fori_loop(0, i, lambda j, c: chunk(j, c, None),
                          (mz, lz, az))
    tri = (jax.lax.broadcasted_iota(jnp.int32, (bq, bq), 1)
           <= jax.lax.broadcasted_iota(jnp.int32, (bq, bq), 0))
    mask2 = jnp.concatenate([tri, tri], axis=0)
    m, l, acc = chunk(i, carry, mask2)
    o = acc / l
    o_ref[...] = jnp.concatenate([o[:bq], o[bq:]], axis=1)


def _attention(qkv, cos, sin, bq=512):
    s = qkv.shape[0]
    nbq = s // bq
    kb = NH * HD // 128
    vb = (NH + NKV) * HD // 128
    return pl.pallas_call(
        functools.partial(_attn_kernel, bq=bq, s=s, scale=1.0 / (HD ** 0.5)),
        grid=(NH // 2, nbq),
        in_specs=[
            pl.BlockSpec((bq, 128), lambda hp, i: (i, hp)),
            pl.BlockSpec((s, 128), lambda hp, i: (0, kb + hp // 2)),
            pl.BlockSpec((s, 128), lambda hp, i: (0, vb + hp // 2)),
            pl.BlockSpec((bq, HD), lambda hp, i: (i, 0)),
            pl.BlockSpec((bq, HD), lambda hp, i: (i, 0)),
            pl.BlockSpec((s, HD), lambda hp, i: (0, 0)),
            pl.BlockSpec((s, HD), lambda hp, i: (0, 0)),
        ],
        out_specs=pl.BlockSpec((bq, 128), lambda hp, i: (i, hp)),
        out_shape=jax.ShapeDtypeStruct((s, NH * HD), jnp.float32),
        scratch_shapes=[pltpu.VMEM((s, HD), jnp.float32),
                        pltpu.VMEM((s, HD), jnp.float32)],
    )(qkv, qkv, qkv, cos, sin, cos, sin)


def _proj_kernel(o_ref, wo_ref, res_ref, w2_ref, wr_ref, h_ref, xn_ref, rl_ref):
    h = res_ref[...] + jnp.dot(o_ref[...], wo_ref[...],
                               preferred_element_type=jnp.float32)
    h_ref[...] = h
    var = jnp.mean(h * h, axis=-1, keepdims=True)
    xn = h * jax.lax.rsqrt(var + EPS) * w2_ref[...]
    xn_ref[...] = xn
    rl_ref[...] = jnp.dot(xn, wr_ref[...], preferred_element_type=jnp.float32)


def _proj_res_norm_router(o, wo, res, ln2_w, wr_pad, bs=256):
    s, d = res.shape
    ncol = wr_pad.shape[1]
    return pl.pallas_call(
        _proj_kernel,
        grid=(s // bs,),
        in_specs=[
            pl.BlockSpec((bs, d), lambda i: (i, 0)),
            pl.BlockSpec((d, d), lambda i: (0, 0)),
            pl.BlockSpec((bs, d), lambda i: (i, 0)),
            pl.BlockSpec((1, d), lambda i: (0, 0)),
            pl.BlockSpec((d, ncol), lambda i: (0, 0)),
        ],
        out_specs=[
            pl.BlockSpec((bs, d), lambda i: (i, 0)),
            pl.BlockSpec((bs, d), lambda i: (i, 0)),
            pl.BlockSpec((bs, ncol), lambda i: (i, 0)),
        ],
        out_shape=[
            jax.ShapeDtypeStruct((s, d), jnp.float32),
            jax.ShapeDtypeStruct((s, d), jnp.float32),
            jax.ShapeDtypeStruct((s, ncol), jnp.float32),
        ],
    )(o, wo, res, ln2_w.reshape(1, d), wr_pad)


_NC = 2
_NS = 16
_NW = _NC * _NS


def _sc_gather_rows(table, idx):
    n = idx.shape[0]
    d = table.shape[1]
    per_w = n // _NW
    chunk = per_w
    for c in (64, 56, 48, 40, 32, 24, 16, 8):
        if per_w % c == 0 and 2 * c * d * 4 + per_w * 4 + 4096 <= 500_000:
            chunk = c
            break
    nch = per_w // chunk
    mesh = plsc.VectorSubcoreMesh(core_axis_name="c", subcore_axis_name="s")

    @functools.partial(
        pl.kernel,
        mesh=mesh,
        out_type=jax.ShapeDtypeStruct((n, d), jnp.float32),
        scratch_types=[
            pltpu.VMEM((per_w,), jnp.int32),
            pltpu.VMEM((chunk, d), jnp.float32),
            pltpu.VMEM((chunk, d), jnp.float32),
            pltpu.SemaphoreType.DMA,
            pltpu.SemaphoreType.DMA,
            pltpu.SemaphoreType.DMA,
            pltpu.SemaphoreType.DMA,
        ],
    )
    def k(table_hbm, idx_hbm, out_hbm, idx_v, buf0, buf1, sg0, sg1, so0, so1):
        wid = lax.axis_index("s") * _NC + lax.axis_index("c")
        base = wid * per_w
        bufs = (buf0, buf1)
        sg = (sg0, sg1)
        so = (so0, so1)
        pltpu.sync_copy(idx_hbm.at[pl.ds(base, per_w)], idx_v)

        def gather(c):
            return pltpu.async_copy(
                table_hbm.at[idx_v.at[pl.ds(c * chunk, chunk)]],
                bufs[c % 2], sg[c % 2])

        def put(c):
            return pltpu.async_copy(
                bufs[c % 2], out_hbm.at[pl.ds(base + c * chunk, chunk)],
                so[c % 2])

        hg = {0: gather(0)}
        ho = {}
        for c in range(nch):
            if c + 1 < nch:
                if c - 1 >= 0:
                    ho[c - 1].wait()
                hg[c + 1] = gather(c + 1)
            hg[c].wait()
            ho[c] = put(c)
        if nch >= 2:
            ho[nch - 2].wait()
        ho[nch - 1].wait()

    return k(table, idx)


def _group_ffn_kernel(meta_ref, xs_ref, w1_ref, w3_ref, w2_ref, ys_ref):
    b = pl.program_id(0)

    @pl.when(meta_ref[1, b] == 1)
    def _():
        x = xs_ref[...]
        a = jnp.dot(x, w1_ref[0], preferred_element_type=jnp.float32)
        c = jnp.dot(x, w3_ref[0], preferred_element_type=jnp.float32)
        g = (a * jax.nn.sigmoid(a)) * c
        ys_ref[...] = jnp.dot(g, w2_ref[0], preferred_element_type=jnp.float32)


def _group_ffn(meta, xs, w1, w3, w2, bt, nblk):
    cap, d = xs.shape
    _, _, dff = w1.shape
    grid_spec = pltpu.PrefetchScalarGridSpec(
        num_scalar_prefetch=1,
        grid=(nblk,),
        in_specs=[
            pl.BlockSpec((bt, d), lambda b, m: (b, 0)),
            pl.BlockSpec((1, d, dff), lambda b, m: (m[0, b], 0, 0)),
            pl.BlockSpec((1, d, dff), lambda b, m: (m[0, b], 0, 0)),
            pl.BlockSpec((1, dff, d), lambda b, m: (m[0, b], 0, 0)),
        ],
        out_specs=pl.BlockSpec((bt, d), lambda b, m: (b, 0)),
    )
    return pl.pallas_call(
        _group_ffn_kernel,
        grid_spec=grid_spec,
        out_shape=jax.ShapeDtypeStruct((cap, d), jnp.float32),
    )(meta, xs, w1, w3, w2)


def _combine_kernel(h2_ref, ya_ref, yb_ref, tw_ref, o_ref):
    wa = tw_ref[:, 0:1]
    wb = tw_ref[:, 1:2]
    o_ref[...] = h2_ref[...] + wa * ya_ref[...] + wb * yb_ref[...]


def _combine(h2, yg, tw_pad, bs=512):
    s, d = h2.shape
    nb = s // bs
    return pl.pallas_call(
        _combine_kernel,
        grid=(nb,),
        in_specs=[
            pl.BlockSpec((bs, d), lambda i: (i, 0)),
            pl.BlockSpec((bs, d), lambda i: (i, 0)),
            pl.BlockSpec((bs, d), lambda i: (nb + i, 0)),
            pl.BlockSpec((bs, 128), lambda i: (i, 0)),
        ],
        out_specs=pl.BlockSpec((bs, d), lambda i: (i, 0)),
        out_shape=jax.ShapeDtypeStruct((s, d), jnp.float32),
    )(h2, yg, yg, tw_pad)


def _moe_kernel(x_ref, dw_ref, h2_ref, w1_ref, w3_ref, w2_ref, o_ref, *, bt):
    e = pl.program_id(0)
    f = pl.program_id(1)
    t = pl.program_id(2)
    x = x_ref[...]
    a = jnp.dot(x, w1_ref[0], preferred_element_type=jnp.float32)
    c = jnp.dot(x, w3_ref[0], preferred_element_type=jnp.float32)
    g = (a * jax.nn.sigmoid(a)) * c
    y = jnp.dot(g, w2_ref[0], preferred_element_type=jnp.float32)
    lane = jax.lax.broadcasted_iota(jnp.int32, dw_ref.shape, 1)
    wcol = jnp.sum(jnp.where(lane == e, dw_ref[...], 0.0), axis=1,
                   keepdims=True)
    contrib = wcol * y
    sl = pl.ds(t * bt, bt)

    @pl.when((e == 0) & (f == 0))
    def _init():
        o_ref[sl, :] = h2_ref[...] + contrib

    @pl.when((e != 0) | (f != 0))
    def _acc():
        o_ref[sl, :] = o_ref[sl, :] + contrib


def _moe(xn, dw_pad, h2, w1, w3, w2, bt=256, bf=1024):
    s, d = xn.shape
    e, _, dff = w1.shape
    ncol = dw_pad.shape[1]
    return pl.pallas_call(
        functools.partial(_moe_kernel, bt=bt),
        grid=(e, dff // bf, s // bt),
        in_specs=[
            pl.BlockSpec((bt, d), lambda ei, f, t: (t, 0)),
            pl.BlockSpec((bt, ncol), lambda ei, f, t: (t, 0)),
            pl.BlockSpec((bt, d), lambda ei, f, t: (t, 0)),
            pl.BlockSpec((1, d, bf), lambda ei, f, t: (ei, 0, f)),
            pl.BlockSpec((1, d, bf), lambda ei, f, t: (ei, 0, f)),
            pl.BlockSpec((1, bf, d), lambda ei, f, t: (ei, f, 0)),
        ],
        out_specs=pl.BlockSpec((s, d), lambda ei, f, t: (0, 0)),
        out_shape=jax.ShapeDtypeStruct((s, d), jnp.float32),
    )(xn, dw_pad, h2, w1, w3, w2)


def kernel(hidden_states, position_ids, gate_logits, ln1_w, ln2_w, wq, wk, wv,
           wo, w_router, w1, w3, w2):
    b, s, d = hidden_states.shape
    x = hidden_states.reshape(s, d)

    wqkv = jnp.concatenate([wq, wk, wv], axis=1)
    qkv = _ln_matmul(x, ln1_w, wqkv)

    pos = position_ids.reshape(s).astype(jnp.float32)
    inv = 1.0 / (THETA ** (jnp.arange(0, HD, 2, dtype=jnp.float32) / HD))
    freqs = pos[:, None] * inv
    emb = jnp.concatenate([freqs, freqs], axis=-1)
    cos = jnp.cos(emb)
    sin = jnp.sin(emb)

    o = _attention(qkv, cos, sin)

    wr_pad = jnp.zeros((d, 128), jnp.float32).at[:, :E].set(w_router)
    h2, xn, rl_pad = _proj_res_norm_router(o, wo, x, ln2_w, wr_pad)
    router_logits = rl_pad[:, :E]

    probs = jax.nn.softmax(router_logits, axis=-1)
    topw, topi = jax.lax.top_k(probs, TOPK)
    topw = topw / jnp.sum(topw, axis=-1, keepdims=True)

    bt = 128
    nblk = (s * TOPK) // bt + E
    cap = nblk * bt
    npair = s * TOPK
    ef = topi.reshape(npair).astype(jnp.int32)
    onehot = (ef[:, None] == jnp.arange(E, dtype=jnp.int32)[None, :]
              ).astype(jnp.int32)
    cum = jnp.cumsum(onehot, axis=0)
    rank = jnp.sum(onehot * cum, axis=1) - 1
    counts = cum[-1]
    blocks = (counts + bt - 1) // bt
    cblocks = jnp.cumsum(blocks).astype(jnp.int32)
    pad_start = bt * jnp.concatenate(
        [jnp.zeros((1,), jnp.int32), cblocks[:-1]])
    pad_pos = pad_start[ef] + rank
    token_of_pair = jnp.arange(npair, dtype=jnp.int32) // TOPK
    src_token = (jnp.arange(cap, dtype=jnp.int32) % s).at[pad_pos].set(
        token_of_pair)
    posAB = jnp.concatenate(
        [pad_pos.reshape(-1, TOPK)[:, 0], pad_pos.reshape(-1, TOPK)[:, 1]])
    nb_used = cblocks[-1]
    bid = jnp.arange(nblk, dtype=jnp.int32)
    bclamp = jnp.minimum(bid, nb_used - 1)
    block_expert = jnp.searchsorted(cblocks, bclamp, side='right')
    meta = jnp.stack([block_expert.astype(jnp.int32),
                      (bid < nb_used).astype(jnp.int32)])

    xs = _sc_gather_rows(xn, src_token)
    ys = _group_ffn(meta, xs, w1, w3, w2, bt, nblk)
    yg = _sc_gather_rows(ys, posAB)
    tw_pad = jnp.zeros((s, 128), jnp.float32).at[:, :TOPK].set(topw)
    h_out = _combine(h2, yg, tw_pad)

    gate_logits = gate_logits.at[LAYER_IDX].set(router_logits)
    return (h_out.reshape(b, s, d), position_ids, gate_logits)


# speedup vs baseline: 1.2244x; 1.0230x over previous
import functools

import jax
import jax.numpy as jnp
from jax import lax
from jax.experimental import pallas as pl
from jax.experimental.pallas import tpu as pltpu
from jax.experimental.pallas import tpu_sc as plsc

D = 1024
NH = 16
NKV = 8
HD = 64
DFF = 2048
E = 8
TOPK = 2
EPS = 1e-6
THETA = 10000.0
LAYER_IDX = 0
NEG = -1e30
_STOP = 0


def _ln_mm_kernel(x_ref, w_ref, wm_ref, o_ref):
    x = x_ref[...]
    var = jnp.mean(x * x, axis=-1, keepdims=True)
    xn = x * jax.lax.rsqrt(var + EPS) * w_ref[...]
    o_ref[...] = jnp.dot(xn, wm_ref[...], preferred_element_type=jnp.float32)


def _ln_matmul(x, w, wm, bs=256):
    s, d = x.shape
    n = wm.shape[1]
    return pl.pallas_call(
        _ln_mm_kernel,
        grid=(s // bs,),
        in_specs=[
            pl.BlockSpec((bs, d), lambda i: (i, 0)),
            pl.BlockSpec((1, d), lambda i: (0, 0)),
            pl.BlockSpec((d, n), lambda i: (0, 0)),
        ],
        out_specs=pl.BlockSpec((bs, n), lambda i: (i, 0)),
        out_shape=jax.ShapeDtypeStruct((s, n), jnp.float32),
    )(x, w.reshape(1, d), wm)


def _rope_apply(x, cos, sin):
    x1 = x[:, :HD // 2]
    x2 = x[:, HD // 2:]
    rot = jnp.concatenate([-x2, x1], axis=-1)
    return x * cos + rot * sin


def _attn_kernel(q_ref, k_ref, v_ref, cq_ref, sq_ref, ck_ref, sk_ref, o_ref,
                 kr_ref, vr_ref, *, bq, s, scale):
    hp = pl.program_id(0)
    i = pl.program_id(1)

    @pl.when(i == 0)
    def _():
        kp = k_ref[...]
        vp = v_ref[...]
        odd = (hp % 2) == 1
        ksel = jnp.where(odd, kp[:, HD:], kp[:, :HD])
        vsel = jnp.where(odd, vp[:, HD:], vp[:, :HD])
        kr_ref[...] = _rope_apply(ksel, ck_ref[...], sk_ref[...])
        vr_ref[...] = vsel

    qp = q_ref[...]
    cq = cq_ref[...]
    sq = sq_ref[...]
    q0 = _rope_apply(qp[:, :HD], cq, sq) * scale
    q1 = _rope_apply(qp[:, HD:], cq, sq) * scale
    q01 = jnp.concatenate([q0, q1], axis=0)

    def chunk(j, carry, mask2):
        m, l, acc = carry
        kj = kr_ref[pl.ds(j * bq, bq), :]
        vj = vr_ref[pl.ds(j * bq, bq), :]
        sc = lax.dot_general(q01, kj, (((1,), (1,)), ((), ())),
                             preferred_element_type=jnp.float32)
        if mask2 is not None:
            sc = jnp.where(mask2, sc, NEG)
        mn = jnp.maximum(m, jnp.max(sc, axis=-1, keepdims=True))
        p = jnp.exp(sc - mn)
        corr = jnp.exp(m - mn)
        l = l * corr + jnp.sum(p, axis=-1, keepdims=True)
        acc = acc * corr + jnp.dot(p, vj, preferred_element_type=jnp.float32)
        return mn, l, acc

    mz = jnp.full((2 * bq, 1), NEG, jnp.float32)
    lz = jnp.zeros((2 * bq, 1), jnp.float32)
    az = jnp.zeros((2 * bq, HD), jnp.float32)
    carry = lax.fori_loop(0, i, lambda j, c: chunk(j, c, None),
                          (mz, lz, az))
    tri = (jax.lax.broadcasted_iota(jnp.int32, (bq, bq), 1)
           <= jax.lax.broadcasted_iota(jnp.int32, (bq, bq), 0))
    mask2 = jnp.concatenate([tri, tri], axis=0)
    m, l, acc = chunk(i, carry, mask2)
    o = acc / l
    o_ref[...] = jnp.concatenate([o[:bq], o[bq:]], axis=1)


def _attention(qkv, cos, sin, bq=512):
    s = qkv.shape[0]
    nbq = s // bq
    kb = NH * HD // 128
    vb = (NH + NKV) * HD // 128
    return pl.pallas_call(
        functools.partial(_attn_kernel, bq=bq, s=s, scale=1.0 / (HD ** 0.5)),
        grid=(NH // 2, nbq),
        in_specs=[
            pl.BlockSpec((bq, 128), lambda hp, i: (i, hp)),
            pl.BlockSpec((s, 128), lambda hp, i: (0, kb + hp // 2)),
            pl.BlockSpec((s, 128), lambda hp, i: (0, vb + hp // 2)),
            pl.BlockSpec((bq, HD), lambda hp, i: (i, 0)),
            pl.BlockSpec((bq, HD), lambda hp, i: (i, 0)),
            pl.BlockSpec((s, HD), lambda hp, i: (0, 0)),
            pl.BlockSpec((s, HD), lambda hp, i: (0, 0)),
        ],
        out_specs=pl.BlockSpec((bq, 128), lambda hp, i: (i, hp)),
        out_shape=jax.ShapeDtypeStruct((s, NH * HD), jnp.float32),
        scratch_shapes=[pltpu.VMEM((s, HD), jnp.float32),
                        pltpu.VMEM((s, HD), jnp.float32)],
    )(qkv, qkv, qkv, cos, sin, cos, sin)


def _proj_kernel(o_ref, wo_ref, res_ref, w2_ref, wr_ref, h_ref, xn_ref, rl_ref):
    h = res_ref[...] + jnp.dot(o_ref[...], wo_ref[...],
                               preferred_element_type=jnp.float32)
    h_ref[...] = h
    var = jnp.mean(h * h, axis=-1, keepdims=True)
    xn = h * jax.lax.rsqrt(var + EPS) * w2_ref[...]
    xn_ref[...] = xn
    rl_ref[...] = jnp.dot(xn, wr_ref[...], preferred_element_type=jnp.float32)


def _proj_res_norm_router(o, wo, res, ln2_w, wr_pad, bs=256):
    s, d = res.shape
    ncol = wr_pad.shape[1]
    return pl.pallas_call(
        _proj_kernel,
        grid=(s // bs,),
        in_specs=[
            pl.BlockSpec((bs, d), lambda i: (i, 0)),
            pl.BlockSpec((d, d), lambda i: (0, 0)),
            pl.BlockSpec((bs, d), lambda i: (i, 0)),
            pl.BlockSpec((1, d), lambda i: (0, 0)),
            pl.BlockSpec((d, ncol), lambda i: (0, 0)),
        ],
        out_specs=[
            pl.BlockSpec((bs, d), lambda i: (i, 0)),
            pl.BlockSpec((bs, d), lambda i: (i, 0)),
            pl.BlockSpec((bs, ncol), lambda i: (i, 0)),
        ],
        out_shape=[
            jax.ShapeDtypeStruct((s, d), jnp.float32),
            jax.ShapeDtypeStruct((s, d), jnp.float32),
            jax.ShapeDtypeStruct((s, ncol), jnp.float32),
        ],
    )(o, wo, res, ln2_w.reshape(1, d), wr_pad)


def _route_kernel(rl_ref, tw_ref, pos_ref, cb_ref, *, bt, s):
    lane = jax.lax.broadcasted_iota(jnp.int32, (s, 128), 1)
    valid = lane < E
    x = jnp.where(valid, rl_ref[...], NEG)
    m1 = jnp.max(x, axis=-1, keepdims=True)
    sel1 = x == m1
    idx1 = jnp.min(jnp.where(sel1, lane, 127), axis=-1, keepdims=True)
    x2 = jnp.where(lane == idx1, NEG, x)
    m2 = jnp.max(x2, axis=-1, keepdims=True)
    sel2 = x2 == m2
    idx2 = jnp.min(jnp.where(sel2, lane, 127), axis=-1, keepdims=True)
    w1 = jax.nn.sigmoid(m1 - m2)
    w2 = jax.nn.sigmoid(m2 - m1)
    a1 = (lane == idx1).astype(jnp.float32)
    a2 = (lane == idx2).astype(jnp.float32)
    c = a1 + a2
    ch = 256
    tril = (jax.lax.broadcasted_iota(jnp.int32, (ch, ch), 0)
            > jax.lax.broadcasted_iota(jnp.int32, (ch, ch), 1)
            ).astype(jnp.float32)
    run = jnp.zeros((1, 128), jnp.float32)
    parts = []
    for i in range(s // ch):
        cc = c[i * ch:(i + 1) * ch]
        pc = jnp.dot(tril, cc, preferred_element_type=jnp.float32) + run
        run = pc[ch - 1:ch] + cc[ch - 1:ch]
        parts.append(pc)
    prefix = jnp.concatenate(parts, axis=0)
    totals = run
    blocks = jnp.floor((totals + (bt - 1)) * (1.0 / bt))
    triu = (jax.lax.broadcasted_iota(jnp.int32, (128, 128), 0)
            <= jax.lax.broadcasted_iota(jnp.int32, (128, 128), 1)
            ).astype(jnp.float32)
    cblocks = jnp.dot(blocks, triu, preferred_element_type=jnp.float32)
    pad_start = bt * (cblocks - blocks)
    r1 = jnp.sum(a1 * prefix, axis=-1, keepdims=True)
    r2 = jnp.sum(a2 * prefix, axis=-1, keepdims=True)
    p1 = jnp.sum(a1 * pad_start, axis=-1, keepdims=True) + r1
    p2 = jnp.sum(a2 * pad_start, axis=-1, keepdims=True) + r2
    lane0 = lane == 0
    lane1 = lane == 1
    tw_ref[...] = jnp.where(lane0, w1, 0.0) + jnp.where(lane1, w2, 0.0)
    pos_ref[...] = (jnp.where(lane0, p1, 0.0)
                    + jnp.where(lane1, p2, 0.0)).astype(jnp.int32)
    cb_ref[...] = cblocks.astype(jnp.int32)


def _route(rl_pad, bt):
    s = rl_pad.shape[0]
    return pl.pallas_call(
        functools.partial(_route_kernel, bt=bt, s=s),
        in_specs=[pl.BlockSpec((s, 128), lambda: (0, 0))],
        out_specs=[
            pl.BlockSpec((s, 128), lambda: (0, 0)),
            pl.BlockSpec((s, 128), lambda: (0, 0)),
            pl.BlockSpec((1, 128), lambda: (0, 0)),
        ],
        out_shape=[
            jax.ShapeDtypeStruct((s, 128), jnp.float32),
            jax.ShapeDtypeStruct((s, 128), jnp.int32),
            jax.ShapeDtypeStruct((1, 128), jnp.int32),
        ],
    )(rl_pad)


_NC = 2
_NS = 16
_NW = _NC * _NS


def _sc_gather_rows(table, idx):
    n = idx.shape[0]
    d = table.shape[1]
    per_w = n // _NW
    chunk = per_w
    for c in (64, 56, 48, 40, 32, 24, 16, 8):
        if per_w % c == 0 and 2 * c * d * 4 + per_w * 4 + 4096 <= 500_000:
            chunk = c
            break
    nch = per_w // chunk
    mesh = plsc.VectorSubcoreMesh(core_axis_name="c", subcore_axis_name="s")

    @functools.partial(
        pl.kernel,
        mesh=mesh,
        out_type=jax.ShapeDtypeStruct((n, d), jnp.float32),
        scratch_types=[
            pltpu.VMEM((per_w,), jnp.int32),
            pltpu.VMEM((chunk, d), jnp.float32),
            pltpu.VMEM((chunk, d), jnp.float32),
            pltpu.SemaphoreType.DMA,
            pltpu.SemaphoreType.DMA,
            pltpu.SemaphoreType.DMA,
            pltpu.SemaphoreType.DMA,
        ],
    )
    def k(table_hbm, idx_hbm, out_hbm, idx_v, buf0, buf1, sg0, sg1, so0, so1):
        wid = lax.axis_index("s") * _NC + lax.axis_index("c")
        base = wid * per_w
        bufs = (buf0, buf1)
        sg = (sg0, sg1)
        so = (so0, so1)
        pltpu.sync_copy(idx_hbm.at[pl.ds(base, per_w)], idx_v)

        def gather(c):
            return pltpu.async_copy(
                table_hbm.at[idx_v.at[pl.ds(c * chunk, chunk)]],
                bufs[c % 2], sg[c % 2])

        def put(c):
            return pltpu.async_copy(
                bufs[c % 2], out_hbm.at[pl.ds(base + c * chunk, chunk)],
                so[c % 2])

        hg = {0: gather(0)}
        ho = {}
        for c in range(nch):
            if c + 1 < nch:
                if c - 1 >= 0:
                    ho[c - 1].wait()
                hg[c + 1] = gather(c + 1)
            hg[c].wait()
            ho[c] = put(c)
        if nch >= 2:
            ho[nch - 2].wait()
        ho[nch - 1].wait()

    return k(table, idx)


def _group_ffn_kernel(meta_ref, xs_ref, w1_ref, w3_ref, w2_ref, ys_ref):
    b = pl.program_id(0)

    @pl.when(meta_ref[1, b] == 1)
    def _():
        x = xs_ref[...]
        a = jnp.dot(x, w1_ref[0], preferred_element_type=jnp.float32)
        c = jnp.dot(x, w3_ref[0], preferred_element_type=jnp.float32)
        g = (a * jax.nn.sigmoid(a)) * c
        ys_ref[...] = jnp.dot(g, w2_ref[0], preferred_element_type=jnp.float32)


def _group_ffn(meta, xs, w1, w3, w2, bt, nblk):
    cap, d = xs.shape
    _, _, dff = w1.shape
    grid_spec = pltpu.PrefetchScalarGridSpec(
        num_scalar_prefetch=1,
        grid=(nblk,),
        in_specs=[
            pl.BlockSpec((bt, d), lambda b, m: (b, 0)),
            pl.BlockSpec((1, d, dff), lambda b, m: (m[0, b], 0, 0)),
            pl.BlockSpec((1, d, dff), lambda b, m: (m[0, b], 0, 0)),
            pl.BlockSpec((1, dff, d), lambda b, m: (m[0, b], 0, 0)),
        ],
        out_specs=pl.BlockSpec((bt, d), lambda b, m: (b, 0)),
    )
    return pl.pallas_call(
        _group_ffn_kernel,
        grid_spec=grid_spec,
        out_shape=jax.ShapeDtypeStruct((cap, d), jnp.float32),
    )(meta, xs, w1, w3, w2)


def _combine_kernel(h2_ref, ya_ref, yb_ref, tw_ref, o_ref):
    wa = tw_ref[:, 0:1]
    wb = tw_ref[:, 1:2]
    o_ref[...] = h2_ref[...] + wa * ya_ref[...] + wb * yb_ref[...]


def _combine(h2, yg, tw_pad, bs=512):
    s, d = h2.shape
    nb = s // bs
    return pl.pallas_call(
        _combine_kernel,
        grid=(nb,),
        in_specs=[
            pl.BlockSpec((bs, d), lambda i: (i, 0)),
            pl.BlockSpec((bs, d), lambda i: (i, 0)),
            pl.BlockSpec((bs, d), lambda i: (nb + i, 0)),
            pl.BlockSpec((bs, 128), lambda i: (i, 0)),
        ],
        out_specs=pl.BlockSpec((bs, d), lambda i: (i, 0)),
        out_shape=jax.ShapeDtypeStruct((s, d), jnp.float32),
    )(h2, yg, yg, tw_pad)


def _moe_kernel(x_ref, dw_ref, h2_ref, w1_ref, w3_ref, w2_ref, o_ref, *, bt):
    e = pl.program_id(0)
    f = pl.program_id(1)
    t = pl.program_id(2)
    x = x_ref[...]
    a = jnp.dot(x, w1_ref[0], preferred_element_type=jnp.float32)
    c = jnp.dot(x, w3_ref[0], preferred_element_type=jnp.float32)
    g = (a * jax.nn.sigmoid(a)) * c
    y = jnp.dot(g, w2_ref[0], preferred_element_type=jnp.float32)
    lane = jax.lax.broadcasted_iota(jnp.int32, dw_ref.shape, 1)
    wcol = jnp.sum(jnp.where(lane == e, dw_ref[...], 0.0), axis=1,
                   keepdims=True)
    contrib = wcol * y
    sl = pl.ds(t * bt, bt)

    @pl.when((e == 0) & (f == 0))
    def _init():
        o_ref[sl, :] = h2_ref[...] + contrib

    @pl.when((e != 0) | (f != 0))
    def _acc():
        o_ref[sl, :] = o_ref[sl, :] + contrib


def _moe(xn, dw_pad, h2, w1, w3, w2, bt=256, bf=1024):
    s, d = xn.shape
    e, _, dff = w1.shape
    ncol = dw_pad.shape[1]
    return pl.pallas_call(
        functools.partial(_moe_kernel, bt=bt),
        grid=(e, dff // bf, s // bt),
        in_specs=[
            pl.BlockSpec((bt, d), lambda ei, f, t: (t, 0)),
            pl.BlockSpec((bt, ncol), lambda ei, f, t: (t, 0)),
            pl.BlockSpec((bt, d), lambda ei, f, t: (t, 0)),
            pl.BlockSpec((1, d, bf), lambda ei, f, t: (ei, 0, f)),
            pl.BlockSpec((1, d, bf), lambda ei, f, t: (ei, 0, f)),
            pl.BlockSpec((1, bf, d), lambda ei, f, t: (ei, f, 0)),
        ],
        out_specs=pl.BlockSpec((s, d), lambda ei, f, t: (0, 0)),
        out_shape=jax.ShapeDtypeStruct((s, d), jnp.float32),
    )(xn, dw_pad, h2, w1, w3, w2)


def kernel(hidden_states, position_ids, gate_logits, ln1_w, ln2_w, wq, wk, wv,
           wo, w_router, w1, w3, w2):
    b, s, d = hidden_states.shape
    x = hidden_states.reshape(s, d)

    wqkv = jnp.concatenate([wq, wk, wv], axis=1)
    qkv = _ln_matmul(x, ln1_w, wqkv)

    pos = position_ids.reshape(s).astype(jnp.float32)
    inv = 1.0 / (THETA ** (jnp.arange(0, HD, 2, dtype=jnp.float32) / HD))
    freqs = pos[:, None] * inv
    emb = jnp.concatenate([freqs, freqs], axis=-1)
    cos = jnp.cos(emb)
    sin = jnp.sin(emb)

    o = _attention(qkv, cos, sin)
    if _STOP == 1:
        return (o.reshape(b, s, d), position_ids, gate_logits)

    wr_pad = jnp.zeros((d, 128), jnp.float32).at[:, :E].set(w_router)
    h2, xn, rl_pad = _proj_res_norm_router(o, wo, x, ln2_w, wr_pad)
    router_logits = rl_pad[:, :E]

    bt = 128
    nblk = (s * TOPK) // bt + E
    cap = nblk * bt
    tw_pad, posout, cbout = _route(rl_pad, bt)
    pos1 = posout[:, 0]
    pos2 = posout[:, 1]
    cblocks = cbout[0, :E]
    tok = jnp.arange(s, dtype=jnp.int32)
    src_token = (jnp.arange(cap, dtype=jnp.int32) % s
                 ).at[pos1].set(tok).at[pos2].set(tok)
    posAB = jnp.concatenate([pos1, pos2])
    nb_used = cblocks[-1]
    bid = jnp.arange(nblk, dtype=jnp.int32)
    bclamp = jnp.minimum(bid, nb_used - 1)
    block_expert = jnp.searchsorted(cblocks, bclamp, side='right')
    meta = jnp.stack([block_expert.astype(jnp.int32),
                      (bid < nb_used).astype(jnp.int32)])

    if _STOP == 2:
        probe = (jnp.sum(meta) + jnp.sum(src_token) + jnp.sum(posAB)
                 ).astype(jnp.float32) + jnp.sum(tw_pad)
        return (h2.reshape(b, s, d) * probe, position_ids, gate_logits)

    xs = _sc_gather_rows(xn, src_token)
    ys = _group_ffn(meta, xs, w1, w3, w2, bt, nblk)
    yg = _sc_gather_rows(ys, posAB)
    h_out = _combine(h2, yg, tw_pad)

    gate_logits = gate_logits.at[LAYER_IDX].set(router_logits)
    return (h_out.reshape(b, s, d), position_ids, gate_logits)


# speedup vs baseline: 1.2958x; 1.0583x over previous
import functools

import jax
import jax.numpy as jnp
from jax import lax
from jax.experimental import pallas as pl
from jax.experimental.pallas import tpu as pltpu
from jax.experimental.pallas import tpu_sc as plsc

D = 1024
NH = 16
NKV = 8
HD = 64
DFF = 2048
E = 8
TOPK = 2
EPS = 1e-6
THETA = 10000.0
LAYER_IDX = 0
NEG = -1e30
_STOP = 0


def _ln_mm_kernel(x_ref, w_ref, wm_ref, o_ref):
    x = x_ref[...]
    var = jnp.mean(x * x, axis=-1, keepdims=True)
    xn = x * jax.lax.rsqrt(var + EPS) * w_ref[...]
    o_ref[...] = jnp.dot(xn, wm_ref[...], preferred_element_type=jnp.float32)


def _ln_matmul(x, w, wm, bs=256):
    s, d = x.shape
    n = wm.shape[1]
    return pl.pallas_call(
        _ln_mm_kernel,
        grid=(s // bs,),
        in_specs=[
            pl.BlockSpec((bs, d), lambda i: (i, 0)),
            pl.BlockSpec((1, d), lambda i: (0, 0)),
            pl.BlockSpec((d, n), lambda i: (0, 0)),
        ],
        out_specs=pl.BlockSpec((bs, n), lambda i: (i, 0)),
        out_shape=jax.ShapeDtypeStruct((s, n), jnp.float32),
    )(x, w.reshape(1, d), wm)


def _rope_apply(x, cos, sin):
    x1 = x[:, :HD // 2]
    x2 = x[:, HD // 2:]
    rot = jnp.concatenate([-x2, x1], axis=-1)
    return x * cos + rot * sin


def _attn_kernel(q_ref, k_ref, v_ref, cq_ref, sq_ref, ck_ref, sk_ref, o_ref,
                 kr_ref, vr_ref, *, bq, s, scale):
    hp = pl.program_id(0)
    i = pl.program_id(1)

    @pl.when(i == 0)
    def _():
        kp = k_ref[...]
        vp = v_ref[...]
        odd = (hp % 2) == 1
        ksel = jnp.where(odd, kp[:, HD:], kp[:, :HD])
        vsel = jnp.where(odd, vp[:, HD:], vp[:, :HD])
        kr_ref[...] = _rope_apply(ksel, ck_ref[...], sk_ref[...])
        sshape = vsel.shape[0]
        vr_ref[...] = jnp.concatenate(
            [vsel, jnp.ones((sshape, 1), jnp.float32),
             jnp.zeros((sshape, 128 - HD - 1), jnp.float32)], axis=1)

    qp = q_ref[...]
    cq = cq_ref[...]
    sq = sq_ref[...]
    q0 = _rope_apply(qp[:, :HD], cq, sq) * scale
    q1 = _rope_apply(qp[:, HD:], cq, sq) * scale
    q01 = jnp.concatenate([q0, q1], axis=0)

    def chunk(j, carry, mask2):
        m, acc = carry
        kj = kr_ref[pl.ds(j * bq, bq), :]
        vj = vr_ref[pl.ds(j * bq, bq), :]
        sc = lax.dot_general(q01, kj, (((1,), (1,)), ((), ())),
                             preferred_element_type=jnp.float32)
        if mask2 is not None:
            sc = jnp.where(mask2, sc, NEG)
        mn = jnp.maximum(m, jnp.max(sc, axis=-1, keepdims=True))
        p = jnp.exp(sc - mn)
        corr = jnp.exp(m - mn)
        acc = acc * corr + jnp.dot(p, vj, preferred_element_type=jnp.float32)
        return mn, acc

    mz = jnp.full((2 * bq, 1), NEG, jnp.float32)
    az = jnp.zeros((2 * bq, 128), jnp.float32)
    carry = lax.fori_loop(0, i, lambda j, c: chunk(j, c, None), (mz, az))
    tri = (jax.lax.broadcasted_iota(jnp.int32, (bq, bq), 1)
           <= jax.lax.broadcasted_iota(jnp.int32, (bq, bq), 0))
    mask2 = jnp.concatenate([tri, tri], axis=0)
    m, acc = chunk(i, carry, mask2)
    o = acc[:, :HD] / acc[:, HD:HD + 1]
    o_ref[...] = jnp.concatenate([o[:bq], o[bq:]], axis=1)


def _attention(qkv, cos, sin, bq=512):
    s = qkv.shape[0]
    nbq = s // bq
    kb = NH * HD // 128
    vb = (NH + NKV) * HD // 128
    return pl.pallas_call(
        functools.partial(_attn_kernel, bq=bq, s=s, scale=1.0 / (HD ** 0.5)),
        grid=(NH // 2, nbq),
        in_specs=[
            pl.BlockSpec((bq, 128), lambda hp, i: (i, hp)),
            pl.BlockSpec((s, 128), lambda hp, i: (0, kb + hp // 2)),
            pl.BlockSpec((s, 128), lambda hp, i: (0, vb + hp // 2)),
            pl.BlockSpec((bq, HD), lambda hp, i: (i, 0)),
            pl.BlockSpec((bq, HD), lambda hp, i: (i, 0)),
            pl.BlockSpec((s, HD), lambda hp, i: (0, 0)),
            pl.BlockSpec((s, HD), lambda hp, i: (0, 0)),
        ],
        out_specs=pl.BlockSpec((bq, 128), lambda hp, i: (i, hp)),
        out_shape=jax.ShapeDtypeStruct((s, NH * HD), jnp.float32),
        scratch_shapes=[pltpu.VMEM((s, HD), jnp.float32),
                        pltpu.VMEM((s, 128), jnp.float32)],
    )(qkv, qkv, qkv, cos, sin, cos, sin)


def _proj_kernel(o_ref, wo_ref, res_ref, w2_ref, wr_ref, h_ref, xn_ref, rl_ref):
    h = res_ref[...] + jnp.dot(o_ref[...], wo_ref[...],
                               preferred_element_type=jnp.float32)
    h_ref[...] = h
    var = jnp.mean(h * h, axis=-1, keepdims=True)
    xn = h * jax.lax.rsqrt(var + EPS) * w2_ref[...]
    xn_ref[...] = xn
    rl_ref[...] = jnp.dot(xn, wr_ref[...], preferred_element_type=jnp.float32)


def _proj_res_norm_router(o, wo, res, ln2_w, wr_pad, bs=256):
    s, d = res.shape
    ncol = wr_pad.shape[1]
    return pl.pallas_call(
        _proj_kernel,
        grid=(s // bs,),
        in_specs=[
            pl.BlockSpec((bs, d), lambda i: (i, 0)),
            pl.BlockSpec((d, d), lambda i: (0, 0)),
            pl.BlockSpec((bs, d), lambda i: (i, 0)),
            pl.BlockSpec((1, d), lambda i: (0, 0)),
            pl.BlockSpec((d, ncol), lambda i: (0, 0)),
        ],
        out_specs=[
            pl.BlockSpec((bs, d), lambda i: (i, 0)),
            pl.BlockSpec((bs, d), lambda i: (i, 0)),
            pl.BlockSpec((bs, ncol), lambda i: (i, 0)),
        ],
        out_shape=[
            jax.ShapeDtypeStruct((s, d), jnp.float32),
            jax.ShapeDtypeStruct((s, d), jnp.float32),
            jax.ShapeDtypeStruct((s, ncol), jnp.float32),
        ],
    )(o, wo, res, ln2_w.reshape(1, d), wr_pad)


def _route_kernel(rl_ref, tw_ref, pos_ref, meta_ref, *, bt, s):
    lane = jax.lax.broadcasted_iota(jnp.int32, (s, 128), 1)
    valid = lane < E
    x = jnp.where(valid, rl_ref[...], NEG)
    m1 = jnp.max(x, axis=-1, keepdims=True)
    sel1 = x == m1
    idx1 = jnp.min(jnp.where(sel1, lane, 127), axis=-1, keepdims=True)
    x2 = jnp.where(lane == idx1, NEG, x)
    m2 = jnp.max(x2, axis=-1, keepdims=True)
    sel2 = x2 == m2
    idx2 = jnp.min(jnp.where(sel2, lane, 127), axis=-1, keepdims=True)
    w1 = jax.nn.sigmoid(m1 - m2)
    w2 = jax.nn.sigmoid(m2 - m1)
    a1 = (lane == idx1).astype(jnp.float32)
    a2 = (lane == idx2).astype(jnp.float32)
    c = a1 + a2
    ch = 256
    tril = (jax.lax.broadcasted_iota(jnp.int32, (ch, ch), 0)
            > jax.lax.broadcasted_iota(jnp.int32, (ch, ch), 1)
            ).astype(jnp.float32)
    run = jnp.zeros((1, 128), jnp.float32)
    parts = []
    for i in range(s // ch):
        cc = c[i * ch:(i + 1) * ch]
        pc = jnp.dot(tril, cc, preferred_element_type=jnp.float32) + run
        run = pc[ch - 1:ch] + cc[ch - 1:ch]
        parts.append(pc)
    prefix = jnp.concatenate(parts, axis=0)
    totals = run
    blocks = jnp.floor((totals + (bt - 1)) * (1.0 / bt))
    triu = (jax.lax.broadcasted_iota(jnp.int32, (128, 128), 0)
            <= jax.lax.broadcasted_iota(jnp.int32, (128, 128), 1)
            ).astype(jnp.float32)
    cblocks = jnp.dot(blocks, triu, preferred_element_type=jnp.float32)
    pad_start = bt * (cblocks - blocks)
    r1 = jnp.sum(a1 * prefix, axis=-1, keepdims=True)
    r2 = jnp.sum(a2 * prefix, axis=-1, keepdims=True)
    p1 = jnp.sum(a1 * pad_start, axis=-1, keepdims=True) + r1
    p2 = jnp.sum(a2 * pad_start, axis=-1, keepdims=True) + r2
    lane0 = lane == 0
    lane1 = lane == 1
    tw_ref[...] = jnp.where(lane0, w1, 0.0) + jnp.where(lane1, w2, 0.0)
    pos_ref[...] = (jnp.where(lane0, p1, 0.0)
                    + jnp.where(lane1, p2, 0.0)).astype(jnp.int32)
    lane8 = jax.lax.broadcasted_iota(jnp.int32, (1, 128), 1)
    eye = (jax.lax.broadcasted_iota(jnp.int32, (128, 128), 0)
           == jax.lax.broadcasted_iota(jnp.int32, (128, 128), 1)
           ).astype(jnp.float32)
    cb_col = lax.dot_general(eye, cblocks, (((1,), (1,)), ((), ())),
                             preferred_element_type=jnp.float32)
    nbu = jnp.sum(jnp.where(lane8 == E - 1, cblocks, 0.0), axis=-1,
                  keepdims=True)
    bclamp = jnp.minimum(lane8.astype(jnp.float32), nbu - 1.0)
    sub = jax.lax.broadcasted_iota(jnp.int32, (128, 128), 0)
    ind = jnp.where(sub < E, (cb_col <= bclamp).astype(jnp.float32), 0.0)
    bexp = jnp.sum(ind, axis=0, keepdims=True)
    vld = (lane8.astype(jnp.float32) < nbu).astype(jnp.float32)
    sub8 = jax.lax.broadcasted_iota(jnp.int32, (8, 128), 0)
    meta_ref[...] = jnp.where(
        sub8 == 0, bexp, jnp.where(sub8 == 1, vld, 0.0)).astype(jnp.int32)


def _route(rl_pad, bt):
    s = rl_pad.shape[0]
    return pl.pallas_call(
        functools.partial(_route_kernel, bt=bt, s=s),
        in_specs=[pl.BlockSpec((s, 128), lambda: (0, 0))],
        out_specs=[
            pl.BlockSpec((s, 128), lambda: (0, 0)),
            pl.BlockSpec((s, 128), lambda: (0, 0)),
            pl.BlockSpec((8, 128), lambda: (0, 0)),
        ],
        out_shape=[
            jax.ShapeDtypeStruct((s, 128), jnp.float32),
            jax.ShapeDtypeStruct((s, 128), jnp.int32),
            jax.ShapeDtypeStruct((8, 128), jnp.int32),
        ],
    )(rl_pad)


_NC = 2
_NS = 16
_NW = _NC * _NS


def _sc_gather_rows(table, idx):
    n = idx.shape[0]
    d = table.shape[1]
    per_w = n // _NW
    chunk = per_w
    for c in (64, 56, 48, 40, 32, 24, 16, 8):
        if per_w % c == 0 and 2 * c * d * 4 + per_w * 4 + 4096 <= 500_000:
            chunk = c
            break
    nch = per_w // chunk
    mesh = plsc.VectorSubcoreMesh(core_axis_name="c", subcore_axis_name="s")

    @functools.partial(
        pl.kernel,
        mesh=mesh,
        out_type=jax.ShapeDtypeStruct((n, d), jnp.float32),
        scratch_types=[
            pltpu.VMEM((per_w,), jnp.int32),
            pltpu.VMEM((chunk, d), jnp.float32),
            pltpu.VMEM((chunk, d), jnp.float32),
            pltpu.SemaphoreType.DMA,
            pltpu.SemaphoreType.DMA,
            pltpu.SemaphoreType.DMA,
            pltpu.SemaphoreType.DMA,
        ],
    )
    def k(table_hbm, idx_hbm, out_hbm, idx_v, buf0, buf1, sg0, sg1, so0, so1):
        wid = lax.axis_index("s") * _NC + lax.axis_index("c")
        base = wid * per_w
        bufs = (buf0, buf1)
        sg = (sg0, sg1)
        so = (so0, so1)
        pltpu.sync_copy(idx_hbm.at[pl.ds(base, per_w)], idx_v)

        def gather(c):
            return pltpu.async_copy(
                table_hbm.at[idx_v.at[pl.ds(c * chunk, chunk)]],
                bufs[c % 2], sg[c % 2])

        def put(c):
            return pltpu.async_copy(
                bufs[c % 2], out_hbm.at[pl.ds(base + c * chunk, chunk)],
                so[c % 2])

        hg = {0: gather(0)}
        ho = {}
        for c in range(nch):
            if c + 1 < nch:
                if c - 1 >= 0:
                    ho[c - 1].wait()
                hg[c + 1] = gather(c + 1)
            hg[c].wait()
            ho[c] = put(c)
        if nch >= 2:
            ho[nch - 2].wait()
        ho[nch - 1].wait()

    return k(table, idx)


def _group_ffn_kernel(meta_ref, xs_ref, w1_ref, w3_ref, w2_ref, ys_ref):
    b = pl.program_id(0)

    @pl.when(meta_ref[1, b] == 1)
    def _():
        x = xs_ref[...]
        a = jnp.dot(x, w1_ref[0], preferred_element_type=jnp.float32)
        c = jnp.dot(x, w3_ref[0], preferred_element_type=jnp.float32)
        g = (a * jax.nn.sigmoid(a)) * c
        ys_ref[...] = jnp.dot(g, w2_ref[0], preferred_element_type=jnp.float32)


def _group_ffn(meta, xs, w1, w3, w2, bt, nblk):
    cap, d = xs.shape
    _, _, dff = w1.shape
    grid_spec = pltpu.PrefetchScalarGridSpec(
        num_scalar_prefetch=1,
        grid=(nblk,),
        in_specs=[
            pl.BlockSpec((bt, d), lambda b, m: (b, 0)),
            pl.BlockSpec((1, d, dff), lambda b, m: (m[0, b], 0, 0)),
            pl.BlockSpec((1, d, dff), lambda b, m: (m[0, b], 0, 0)),
            pl.BlockSpec((1, dff, d), lambda b, m: (m[0, b], 0, 0)),
        ],
        out_specs=pl.BlockSpec((bt, d), lambda b, m: (b, 0)),
    )
    return pl.pallas_call(
        _group_ffn_kernel,
        grid_spec=grid_spec,
        out_shape=jax.ShapeDtypeStruct((cap, d), jnp.float32),
    )(meta, xs, w1, w3, w2)


def _combine_kernel(h2_ref, ya_ref, yb_ref, tw_ref, o_ref):
    wa = tw_ref[:, 0:1]
    wb = tw_ref[:, 1:2]
    o_ref[...] = h2_ref[...] + wa * ya_ref[...] + wb * yb_ref[...]


def _combine(h2, yg, tw_pad, bs=512):
    s, d = h2.shape
    nb = s // bs
    return pl.pallas_call(
        _combine_kernel,
        grid=(nb,),
        in_specs=[
            pl.BlockSpec((bs, d), lambda i: (i, 0)),
            pl.BlockSpec((bs, d), lambda i: (i, 0)),
            pl.BlockSpec((bs, d), lambda i: (nb + i, 0)),
            pl.BlockSpec((bs, 128), lambda i: (i, 0)),
        ],
        out_specs=pl.BlockSpec((bs, d), lambda i: (i, 0)),
        out_shape=jax.ShapeDtypeStruct((s, d), jnp.float32),
    )(h2, yg, yg, tw_pad)


def _moe_kernel(x_ref, dw_ref, h2_ref, w1_ref, w3_ref, w2_ref, o_ref, *, bt):
    e = pl.program_id(0)
    f = pl.program_id(1)
    t = pl.program_id(2)
    x = x_ref[...]
    a = jnp.dot(x, w1_ref[0], preferred_element_type=jnp.float32)
    c = jnp.dot(x, w3_ref[0], preferred_element_type=jnp.float32)
    g = (a * jax.nn.sigmoid(a)) * c
    y = jnp.dot(g, w2_ref[0], preferred_element_type=jnp.float32)
    lane = jax.lax.broadcasted_iota(jnp.int32, dw_ref.shape, 1)
    wcol = jnp.sum(jnp.where(lane == e, dw_ref[...], 0.0), axis=1,
                   keepdims=True)
    contrib = wcol * y
    sl = pl.ds(t * bt, bt)

    @pl.when((e == 0) & (f == 0))
    def _init():
        o_ref[sl, :] = h2_ref[...] + contrib

    @pl.when((e != 0) | (f != 0))
    def _acc():
        o_ref[sl, :] = o_ref[sl, :] + contrib


def _moe(xn, dw_pad, h2, w1, w3, w2, bt=256, bf=1024):
    s, d = xn.shape
    e, _, dff = w1.shape
    ncol = dw_pad.shape[1]
    return pl.pallas_call(
        functools.partial(_moe_kernel, bt=bt),
        grid=(e, dff // bf, s // bt),
        in_specs=[
            pl.BlockSpec((bt, d), lambda ei, f, t: (t, 0)),
            pl.BlockSpec((bt, ncol), lambda ei, f, t: (t, 0)),
            pl.BlockSpec((bt, d), lambda ei, f, t: (t, 0)),
            pl.BlockSpec((1, d, bf), lambda ei, f, t: (ei, 0, f)),
            pl.BlockSpec((1, d, bf), lambda ei, f, t: (ei, 0, f)),
            pl.BlockSpec((1, bf, d), lambda ei, f, t: (ei, f, 0)),
        ],
        out_specs=pl.BlockSpec((s, d), lambda ei, f, t: (0, 0)),
        out_shape=jax.ShapeDtypeStruct((s, d), jnp.float32),
    )(xn, dw_pad, h2, w1, w3, w2)


def kernel(hidden_states, position_ids, gate_logits, ln1_w, ln2_w, wq, wk, wv,
           wo, w_router, w1, w3, w2):
    b, s, d = hidden_states.shape
    x = hidden_states.reshape(s, d)

    wqkv = jnp.concatenate([wq, wk, wv], axis=1)
    qkv = _ln_matmul(x, ln1_w, wqkv)

    pos = position_ids.reshape(s).astype(jnp.float32)
    inv = 1.0 / (THETA ** (jnp.arange(0, HD, 2, dtype=jnp.float32) / HD))
    freqs = pos[:, None] * inv
    emb = jnp.concatenate([freqs, freqs], axis=-1)
    cos = jnp.cos(emb)
    sin = jnp.sin(emb)

    o = _attention(qkv, cos, sin)
    if _STOP == 1:
        return (o.reshape(b, s, d), position_ids, gate_logits)

    wr_pad = jnp.zeros((d, 128), jnp.float32).at[:, :E].set(w_router)
    h2, xn, rl_pad = _proj_res_norm_router(o, wo, x, ln2_w, wr_pad)
    router_logits = rl_pad[:, :E]

    bt = 128
    nblk = (s * TOPK) // bt + E
    cap = nblk * bt
    tw_pad, posout, metaout = _route(rl_pad, bt)
    pos1 = posout[:, 0]
    pos2 = posout[:, 1]
    meta = metaout[:2, :nblk]
    tok = jnp.arange(s, dtype=jnp.int32)
    src_token = (jnp.arange(cap, dtype=jnp.int32) % s
                 ).at[pos1].set(tok).at[pos2].set(tok)
    posAB = jnp.concatenate([pos1, pos2])

    if _STOP == 2:
        probe = (jnp.sum(meta) + jnp.sum(src_token) + jnp.sum(posAB)
                 ).astype(jnp.float32) + jnp.sum(tw_pad)
        return (h2.reshape(b, s, d) * probe, position_ids, gate_logits)

    xs = _sc_gather_rows(xn, src_token)
    ys = _group_ffn(meta, xs, w1, w3, w2, bt, nblk)
    yg = _sc_gather_rows(ys, posAB)
    h_out = _combine(h2, yg, tw_pad)

    gate_logits = gate_logits.at[LAYER_IDX].set(router_logits)
    return (h_out.reshape(b, s, d), position_ids, gate_logits)
